# baseline (speedup 1.0000x reference)
import jax
import jax.numpy as jnp
from jax import lax
from jax.experimental import pallas as pl
from jax.experimental.pallas import tpu as pltpu
from jax.experimental.pallas import tpu_sc as plsc

VOCAB = 1000000
HIDDEN = 64
B = 4096
L = 200

NC = 2
NS = 16
NW = NC * NS
TOKENS = B * L
PER_W = TOKENS // NW
NCH = PER_W // (2 * L)
HALF = L
OUT_ROWS = TOKENS // 2


def _gather_body(ids_hbm, tl_hbm, out_hbm, idx_v, e0, o0, e1, o1,
                 sg0, sg1, sw0, sw1):
    wid = lax.axis_index("s") * NC + lax.axis_index("c")
    ebufs = (e0, e1)
    obufs = (o0, o1)
    gsems = (sg0, sg1)
    wsems = (sw0, sw1)
    tbase = wid * PER_W
    obase = wid * (PER_W // 2)

    pltpu.sync_copy(ids_hbm.at[pl.ds(tbase, PER_W)], idx_v)

    def gat_e(c, b):
        return pltpu.make_async_copy(
            tl_hbm.at[idx_v.at[pl.ds(c * 2 * L, HALF)]], ebufs[b], gsems[b]
        )

    def gat_o(c, b):
        return pltpu.make_async_copy(
            tl_hbm.at[idx_v.at[pl.ds(c * 2 * L + HALF, HALF)]], obufs[b], gsems[b]
        )

    def wb_e(c, b):
        return pltpu.make_async_copy(
            ebufs[b].at[:, pl.ds(0, HIDDEN)],
            out_hbm.at[pl.ds(obase + c * HALF, HALF), pl.ds(0, HIDDEN)],
            wsems[b],
        )

    def wb_o(c, b):
        return pltpu.make_async_copy(
            obufs[b].at[:, pl.ds(HIDDEN, HIDDEN)],
            out_hbm.at[pl.ds(obase + c * HALF, HALF), pl.ds(HIDDEN, HIDDEN)],
            wsems[b],
        )

    for b in range(2):
        gat_e(b, b).start()
        gat_o(b, b).start()

    def step(g):
        for b in range(2):
            c = g + b
            gat_e(c, b).wait()
            gat_o(c, b).wait()
            wb_e(c, b).start()
            wb_o(c, b).start()
        for b in range(2):
            nxt = g + b + 2

            @pl.when(nxt < NCH)
            def _():
                wb_e(g + b, b).wait()
                wb_o(g + b, b).wait()
                gat_e(nxt, b).start()
                gat_o(nxt, b).start()

    pl.loop(0, NCH, step=2)(step)

    for b in range(2):
        wb_e(NCH - 2 + b, b).wait()
        wb_o(NCH - 2 + b, b).wait()


@jax.jit
def _embed(ids, tl):
    out2 = pl.kernel(
        _gather_body,
        out_type=jax.ShapeDtypeStruct((OUT_ROWS, 128), jnp.float32),
        mesh=plsc.VectorSubcoreMesh(
            core_axis_name="c", subcore_axis_name="s",
            num_cores=NC, num_subcores=NS,
        ),
        scratch_types=[
            pltpu.VMEM((PER_W,), jnp.int32),
            pltpu.VMEM((HALF, 128), jnp.float32),
            pltpu.VMEM((HALF, 128), jnp.float32),
            pltpu.VMEM((HALF, 128), jnp.float32),
            pltpu.VMEM((HALF, 128), jnp.float32),
            pltpu.SemaphoreType.DMA,
            pltpu.SemaphoreType.DMA,
            pltpu.SemaphoreType.DMA,
            pltpu.SemaphoreType.DMA,
        ],
        compiler_params=pltpu.CompilerParams(use_tc_tiling_on_sc=False),
    )(ids, tl)
    return jnp.reshape(out2, (B, L, HIDDEN))


def kernel(input_ids, table):
    tl = jnp.concatenate([table, table], axis=1)
    ids1 = jnp.reshape(input_ids.astype(jnp.int32), (NW * NCH, HALF, 2))
    ids1 = jnp.reshape(jnp.transpose(ids1, (0, 2, 1)), (TOKENS,))
    return _embed(ids1, tl)


# speedup vs baseline: 1.0008x; 1.0008x over previous
import jax
import jax.numpy as jnp
from jax import lax
from jax.experimental import pallas as pl
from jax.experimental.pallas import tpu as pltpu
from jax.experimental.pallas import tpu_sc as plsc

VOCAB = 1000000
HIDDEN = 64
B = 4096
L = 200

NC = 2
NS = 16
NW = NC * NS
TOKENS = B * L
PER_W = TOKENS // NW
NCH = PER_W // (2 * L)
HALF = L
OUT_ROWS = TOKENS // 2


def _gather_body(ids_hbm, tl_hbm, out_hbm, idx_v, e0, o0, e1, o1,
                 sg0, sg1, sw0, sw1):
    wid = lax.axis_index("s") * NC + lax.axis_index("c")
    ebufs = (e0, e1)
    obufs = (o0, o1)
    gsems = (sg0, sg1)
    wsems = (sw0, sw1)
    tbase = wid * PER_W
    obase = wid * (PER_W // 2)

    pltpu.sync_copy(ids_hbm.at[pl.ds(tbase, PER_W)], idx_v)

    def gat_e(c, b):
        return pltpu.make_async_copy(
            tl_hbm.at[idx_v.at[pl.ds(c * 2 * L, HALF)]], ebufs[b], gsems[b]
        )

    def gat_o(c, b):
        return pltpu.make_async_copy(
            tl_hbm.at[idx_v.at[pl.ds(c * 2 * L + HALF, HALF)]], obufs[b], gsems[b]
        )

    def wb_e(c, b):
        return pltpu.make_async_copy(
            ebufs[b].at[:, pl.ds(0, HIDDEN)],
            out_hbm.at[pl.ds(obase + c * HALF, HALF), pl.ds(0, HIDDEN)],
            wsems[b],
        )

    def wb_o(c, b):
        return pltpu.make_async_copy(
            obufs[b].at[:, pl.ds(HIDDEN, HIDDEN)],
            out_hbm.at[pl.ds(obase + c * HALF, HALF), pl.ds(HIDDEN, HIDDEN)],
            wsems[b],
        )

    for b in range(2):
        gat_e(b, b).start()
        gat_o(b, b).start()

    def step(g):
        for b in range(2):
            c = g + b
            gat_e(c, b).wait()
            gat_o(c, b).wait()
            wb_e(c, b).start()
            wb_o(c, b).start()
        for b in range(2):
            nxt = g + b + 2

            @pl.when(nxt < NCH)
            def _():
                wb_e(g + b, b).wait()
                wb_o(g + b, b).wait()
                gat_e(nxt, b).start()
                gat_o(nxt, b).start()

    pl.loop(0, NCH, step=2)(step)

    for b in range(2):
        wb_e(NCH - 2 + b, b).wait()
        wb_o(NCH - 2 + b, b).wait()


@jax.jit
def _embed(ids, tl):
    out2 = pl.kernel(
        _gather_body,
        out_type=jax.ShapeDtypeStruct((OUT_ROWS, 128), jnp.float32),
        mesh=plsc.VectorSubcoreMesh(
            core_axis_name="c", subcore_axis_name="s",
            num_cores=NC, num_subcores=NS,
        ),
        scratch_types=[
            pltpu.VMEM((PER_W,), jnp.int32),
            pltpu.VMEM((HALF, 128), jnp.float32),
            pltpu.VMEM((HALF, 128), jnp.float32),
            pltpu.VMEM((HALF, 128), jnp.float32),
            pltpu.VMEM((HALF, 128), jnp.float32),
            pltpu.SemaphoreType.DMA,
            pltpu.SemaphoreType.DMA,
            pltpu.SemaphoreType.DMA,
            pltpu.SemaphoreType.DMA,
        ],
        compiler_params=pltpu.CompilerParams(use_tc_tiling_on_sc=False, skip_device_barrier=True),
    )(ids, tl)
    return jnp.reshape(out2, (B, L, HIDDEN))


def kernel(input_ids, table):
    tl = jnp.concatenate([table, table], axis=1)
    ids1 = jnp.reshape(input_ids.astype(jnp.int32), (NW * NCH, HALF, 2))
    ids1 = jnp.reshape(jnp.transpose(ids1, (0, 2, 1)), (TOKENS,))
    return _embed(ids1, tl)


# speedup vs baseline: 1.3063x; 1.3052x over previous
import jax
import jax.numpy as jnp
from jax import lax
from jax.experimental import pallas as pl
from jax.experimental.pallas import tpu as pltpu
from jax.experimental.pallas import tpu_sc as plsc

VOCAB = 1000000
HIDDEN = 64
B = 4096
L = 200

NC = 2
NS = 16
NW = NC * NS
TOKENS = B * L
PER_W = TOKENS // NW
CHUNK = 512
NCHUNKS = PER_W // CHUNK
NBUF = 2

assert TOKENS == NW * NCHUNKS * CHUNK


def _body(ids_hbm, table_hbm, out_hbm, idx_v, r0, r1, sg0, sg1, sw0, sw1):
    wid = lax.axis_index("s") * NC + lax.axis_index("c")
    rows = (r0, r1)
    gsems = (sg0, sg1)
    wsems = (sw0, sw1)
    wbase = wid * PER_W

    pltpu.sync_copy(ids_hbm.at[wid], idx_v)

    def gat(c, b):
        return pltpu.make_async_copy(
            table_hbm.at[idx_v.at[c]], rows[b], gsems[b]
        )

    def wb(c, b):
        return pltpu.make_async_copy(
            rows[b], out_hbm.at[pl.ds(wbase + c * CHUNK, CHUNK)], wsems[b]
        )

    for b in range(NBUF):
        gat(b, b).start()

    def step(g):
        for b in range(NBUF):
            c = g + b
            gat(c, b).wait()
            wb(c, b).start()
        for b in range(NBUF):
            c = g + b
            nxt = c + NBUF

            @pl.when(nxt < NCHUNKS)
            def _():
                wb(c, b).wait()
                gat(nxt, b).start()

    pl.loop(0, NCHUNKS, step=NBUF)(step)

    for b in range(NBUF):
        c = NCHUNKS - NBUF + b
        wb(c, b).wait()


@jax.jit
def _embed(ids, table):
    grid_kernel = pl.kernel(
        _body,
        out_type=jax.ShapeDtypeStruct((TOKENS, HIDDEN), jnp.float32),
        mesh=plsc.VectorSubcoreMesh(
            core_axis_name="c", subcore_axis_name="s",
            num_cores=NC, num_subcores=NS,
        ),
        scratch_types=[
            pltpu.VMEM((NCHUNKS, CHUNK), jnp.int32),
            pltpu.VMEM((CHUNK, HIDDEN), jnp.float32),
            pltpu.VMEM((CHUNK, HIDDEN), jnp.float32),
            pltpu.SemaphoreType.DMA,
            pltpu.SemaphoreType.DMA,
            pltpu.SemaphoreType.DMA,
            pltpu.SemaphoreType.DMA,
        ],
        compiler_params=pltpu.CompilerParams(use_tc_tiling_on_sc=False),
    )
    return grid_kernel(ids, table)


def kernel(input_ids, table):
    ids = jnp.reshape(input_ids.astype(jnp.int32), (NW, NCHUNKS, CHUNK))
    out = _embed(ids, table)
    return jnp.reshape(out, (B, L, HIDDEN))
